# baseline (speedup 1.0000x reference)
import functools

import jax
import jax.numpy as jnp
from jax import lax
from jax.experimental import pallas as pl
from jax.experimental.pallas import tpu as pltpu
from jax.experimental.pallas import tpu_sc as plsc

NC = 2
NS = 16
NW = NC * NS
L = 16
K = 128
D = 128
BN = 1024


def _mesh():
    return plsc.VectorSubcoreMesh(
        core_axis_name="c", subcore_axis_name="s", num_cores=NC, num_subcores=NS
    )


def _sc_params():
    return pltpu.CompilerParams(needs_layout_passes=False)


@functools.cache
def _make_deg(n_pad: int, e_pad: int):
    ep = e_pad // NW

    def body(s0, d0, s1, d1, zvec, out, idx, h0, h1, h2, h3):
        c = lax.axis_index("c")
        s = lax.axis_index("s")
        w = s * NC + c
        for h in (h0, h1, h2, h3):
            pltpu.sync_copy(zvec, h)
        ones = jnp.ones((L,), jnp.float32)
        for j, (arr, h) in enumerate(((s0, h0), (d0, h1), (s1, h2), (d1, h3))):
            def chunk(i, carry, arr=arr, h=h):
                pltpu.sync_copy(arr.at[pl.ds(w * ep + i * K, K)], idx)
                for g in range(K // L):
                    iv = idx[pl.ds(g * L, L)]
                    plsc.addupdate_scatter(h, [iv], ones)
                return carry
            lax.fori_loop(0, ep // K, chunk, 0)
            pltpu.sync_copy(h, out.at[j, w])

    return pl.kernel(
        body,
        out_type=jax.ShapeDtypeStruct((4, NW, n_pad), jnp.float32),
        mesh=_mesh(),
        compiler_params=_sc_params(),
        scratch_types=[
            pltpu.VMEM((K,), jnp.int32),
            pltpu.VMEM((n_pad,), jnp.float32),
            pltpu.VMEM((n_pad,), jnp.float32),
            pltpu.VMEM((n_pad,), jnp.float32),
            pltpu.VMEM((n_pad,), jnp.float32),
        ],
    )


def _split_chunks(total_chunks_per_pair: int, frac: float):
    n0 = int(round(total_chunks_per_pair * frac / 2)) * 2
    n0 = max(2, min(total_chunks_per_pair - 2, n0))
    return n0, total_chunks_per_pair - n0


@functools.cache
def _make_conv2(n_pad: int, e_pad: int):
    tch = e_pad // (NS * K)
    n0, n1 = _split_chunks(tch, 0.675)
    np0 = n0 // 2
    rpt = n_pad // NS

    def body(xs0, xs1, src0, dst0, src1, dst1, zrows, out,
             s_a, d_a, s_b, d_b, rows_a, rows_b, acc,
             isem_a, isem_b, gsem_a, gsem_b):
        c = lax.axis_index("c")
        s = lax.axis_index("s")

        def wait_idx(sb, db, sem, src):
            pltpu.make_async_copy(src.at[pl.ds(0, K)], sb, sem).wait()
            pltpu.make_async_copy(src.at[pl.ds(0, K)], db, sem).wait()

        def phase(r, xs, src, dst):
            plsc.subcore_barrier()

            @pl.when(c == 0)
            def _():
                base = s * (n0 * K)
                pltpu.sync_copy(src.at[pl.ds(base, K)], s_a)
                pltpu.sync_copy(dst.at[pl.ds(base, K)], d_a)
                pltpu.async_copy(xs.at[s_a], rows_a, gsem_a)
                pltpu.async_copy(src.at[pl.ds(base + K, K)], s_b, isem_b)
                pltpu.async_copy(dst.at[pl.ds(base + K, K)], d_b, isem_b)

                def pair(p, carry):
                    off2 = base + 2 * p * K
                    wait_idx(s_b, d_b, isem_b, src)
                    pltpu.async_copy(xs.at[s_b], rows_b, gsem_b)
                    pltpu.make_async_copy(
                        xs.at[pl.ds(0, K)], rows_a, gsem_a).wait()
                    pltpu.sync_copy(rows_a, acc.at[d_a], add=True)

                    @pl.when(p < np0 - 1)
                    def _():
                        pltpu.async_copy(src.at[pl.ds(off2 + 2 * K, K)],
                                         s_a, isem_a)
                        pltpu.async_copy(dst.at[pl.ds(off2 + 2 * K, K)],
                                         d_a, isem_a)

                    pltpu.make_async_copy(
                        xs.at[pl.ds(0, K)], rows_b, gsem_b).wait()

                    @pl.when(p < np0 - 1)
                    def _():
                        wait_idx(s_a, d_a, isem_a, src)
                        pltpu.async_copy(xs.at[s_a], rows_a, gsem_a)

                    pltpu.sync_copy(rows_b, acc.at[d_b], add=True)

                    @pl.when(p < np0 - 1)
                    def _():
                        pltpu.async_copy(src.at[pl.ds(off2 + 3 * K, K)],
                                         s_b, isem_b)
                        pltpu.async_copy(dst.at[pl.ds(off2 + 3 * K, K)],
                                         d_b, isem_b)

                    return carry

                lax.fori_loop(0, np0, pair, 0)

            @pl.when(c == 1)
            def _():
                base = NS * n0 * K + s * (n1 * K)

                def chunk(i, carry):
                    off = base + i * K
                    pltpu.sync_copy(src.at[pl.ds(off, K)], s_a)
                    pltpu.sync_copy(dst.at[pl.ds(off, K)], d_a)
                    pltpu.async_copy(xs.at[s_a], rows_a, gsem_a).wait()
                    pltpu.sync_copy(rows_a, acc.at[d_a], add=True)
                    return carry

                lax.fori_loop(0, n1, chunk, 0)

            plsc.subcore_barrier()
            pltpu.sync_copy(acc.at[pl.ds(s * rpt, rpt)],
                            out.at[r, c, pl.ds(s * rpt, rpt)])
            if r == 0:
                pltpu.sync_copy(zrows, acc.at[pl.ds(s * rpt, rpt)])

        pltpu.sync_copy(zrows, acc.at[pl.ds(s * rpt, rpt)])
        phase(0, xs0, src0, dst0)
        phase(1, xs1, src1, dst1)

    return pl.kernel(
        body,
        out_type=jax.ShapeDtypeStruct((2, NC, n_pad, D), jnp.float32),
        mesh=_mesh(),
        compiler_params=_sc_params(),
        scratch_types=[
            pltpu.VMEM((K,), jnp.int32),
            pltpu.VMEM((K,), jnp.int32),
            pltpu.VMEM((K,), jnp.int32),
            pltpu.VMEM((K,), jnp.int32),
            pltpu.VMEM((K, D), jnp.float32),
            pltpu.VMEM((K, D), jnp.float32),
            pltpu.VMEM_SHARED((n_pad, D), jnp.float32),
            pltpu.SemaphoreType.DMA,
            pltpu.SemaphoreType.DMA,
            pltpu.SemaphoreType.DMA,
            pltpu.SemaphoreType.DMA,
        ],
    )


@functools.cache
def _make_score(n_pad: int, e_pad: int):
    tch = e_pad // (NS * K)
    m0, m1 = _split_chunks(tch, 0.75)
    if tch % 8 == 0:
        m0 = max(8, (m0 // 8) * 8)
        m1 = tch - m0
    np0 = m0 // 2

    def body(h, ps, pd, qs, qd, out, s_a, d_a, s_b, d_b,
             srows_a, drows_a, srows_b, drows_b, p16, dots,
             isem_a, isem_b, gs_a, gd_a, gs_b, gd_b):
        c = lax.axis_index("c")
        s = lax.axis_index("s")
        lane = lax.iota(jnp.int32, L)

        def compute(ci, srows, drows):
            def row(r, carry2):
                acc = srows[r, pl.ds(0, L)] * drows[r, pl.ds(0, L)]
                for jj in range(1, D // L):
                    acc = acc + (srows[r, pl.ds(jj * L, L)]
                                 * drows[r, pl.ds(jj * L, L)])
                p16[r, :] = acc
                return carry2

            lax.fori_loop(0, K, row, 0)
            for g in range(K // L):
                rowi = lane + (g * L)
                dv = plsc.load_gather(
                    p16, [rowi, jnp.zeros((L,), jnp.int32)])
                for col in range(1, L):
                    dv = dv + plsc.load_gather(
                        p16, [rowi, jnp.full((L,), col, jnp.int32)])
                dots[ci, pl.ds(g * L, L)] = dv

        def wait_idx(sb, db, sem, sa):
            pltpu.make_async_copy(sa.at[pl.ds(0, K)], sb, sem).wait()
            pltpu.make_async_copy(sa.at[pl.ds(0, K)], db, sem).wait()

        def wait_rows(srows, drows, sem_s, sem_d):
            pltpu.make_async_copy(h.at[pl.ds(0, K)], srows, sem_s).wait()
            pltpu.make_async_copy(h.at[pl.ds(0, K)], drows, sem_d).wait()

        for which, (sa, da) in enumerate(((ps, pd), (qs, qd))):

            @pl.when(c == 0)
            def _(sa=sa, da=da, which=which):
                base = s * (m0 * K)
                pltpu.sync_copy(sa.at[pl.ds(base, K)], s_a)
                pltpu.sync_copy(da.at[pl.ds(base, K)], d_a)
                pltpu.async_copy(h.at[s_a], srows_a, gs_a)
                pltpu.async_copy(h.at[d_a], drows_a, gd_a)
                pltpu.async_copy(sa.at[pl.ds(base + K, K)], s_b, isem_b)
                pltpu.async_copy(da.at[pl.ds(base + K, K)], d_b, isem_b)

                def pair(p, carry):
                    c0 = 2 * p
                    off2 = base + 2 * p * K
                    wait_idx(s_b, d_b, isem_b, sa)
                    pltpu.async_copy(h.at[s_b], srows_b, gs_b)
                    pltpu.async_copy(h.at[d_b], drows_b, gd_b)
                    wait_rows(srows_a, drows_a, gs_a, gd_a)
                    compute(c0, srows_a, drows_a)

                    @pl.when(p < np0 - 1)
                    def _():
                        pltpu.async_copy(sa.at[pl.ds(off2 + 2 * K, K)],
                                         s_a, isem_a)
                        pltpu.async_copy(da.at[pl.ds(off2 + 2 * K, K)],
                                         d_a, isem_a)

                    wait_rows(srows_b, drows_b, gs_b, gd_b)

                    @pl.when(p < np0 - 1)
                    def _():
                        wait_idx(s_a, d_a, isem_a, sa)
                        pltpu.async_copy(h.at[s_a], srows_a, gs_a)
                        pltpu.async_copy(h.at[d_a], drows_a, gd_a)

                    compute(c0 + 1, srows_b, drows_b)

                    @pl.when(p < np0 - 1)
                    def _():
                        pltpu.async_copy(sa.at[pl.ds(off2 + 3 * K, K)],
                                         s_b, isem_b)
                        pltpu.async_copy(da.at[pl.ds(off2 + 3 * K, K)],
                                         d_b, isem_b)

                    return carry

                lax.fori_loop(0, np0, pair, 0)
                cb = pl.multiple_of(s * m0, 8)
                pltpu.sync_copy(dots.at[pl.ds(0, m0)],
                                out.at[which, pl.ds(cb, m0)])

            @pl.when(c == 1)
            def _(sa=sa, da=da, which=which):
                base = NS * m0 * K + s * (m1 * K)

                def chunk(i, carry):
                    off = base + i * K
                    pltpu.sync_copy(sa.at[pl.ds(off, K)], s_a)
                    pltpu.sync_copy(da.at[pl.ds(off, K)], d_a)
                    cp1 = pltpu.async_copy(h.at[s_a], srows_a, gs_a)
                    cp2 = pltpu.async_copy(h.at[d_a], drows_a, gd_a)
                    cp1.wait()
                    cp2.wait()
                    compute(i, srows_a, drows_a)
                    return carry

                lax.fori_loop(0, m1, chunk, 0)
                cb = pl.multiple_of(NS * m0 + s * m1, 8)
                pltpu.sync_copy(dots.at[pl.ds(0, m1)],
                                out.at[which, pl.ds(cb, m1)])

    return pl.kernel(
        body,
        out_type=jax.ShapeDtypeStruct((2, e_pad // K, K), jnp.float32),
        mesh=_mesh(),
        compiler_params=_sc_params(),
        scratch_types=[
            pltpu.VMEM((K,), jnp.int32),
            pltpu.VMEM((K,), jnp.int32),
            pltpu.VMEM((K,), jnp.int32),
            pltpu.VMEM((K,), jnp.int32),
            pltpu.VMEM((K, D), jnp.float32),
            pltpu.VMEM((K, D), jnp.float32),
            pltpu.VMEM((K, D), jnp.float32),
            pltpu.VMEM((K, D), jnp.float32),
            pltpu.VMEM((K, L), jnp.float32),
            pltpu.VMEM((m0, K), jnp.float32),
            pltpu.SemaphoreType.DMA,
            pltpu.SemaphoreType.DMA,
            pltpu.SemaphoreType.DMA,
            pltpu.SemaphoreType.DMA,
            pltpu.SemaphoreType.DMA,
            pltpu.SemaphoreType.DMA,
        ],
    )


def _prep_body(x_ref, degp_ref, xs0_ref, xs1_ref, norms_ref):
    deg = jnp.sum(degp_ref[...], axis=1)
    norms = jnp.where(deg > 0, lax.rsqrt(jnp.maximum(deg, 1.0)), 0.0)
    norms_ref[...] = norms
    x = x_ref[...]
    xs0_ref[...] = x * norms[0][:, None]
    xs1_ref[...] = x * norms[2][:, None]


@functools.cache
def _make_prep(n_pad: int):
    g = n_pad // BN
    return pl.pallas_call(
        _prep_body,
        grid=(g,),
        in_specs=[
            pl.BlockSpec((BN, D), lambda i: (i, 0)),
            pl.BlockSpec((4, NW, BN), lambda i: (0, 0, i)),
        ],
        out_specs=[
            pl.BlockSpec((BN, D), lambda i: (i, 0)),
            pl.BlockSpec((BN, D), lambda i: (i, 0)),
            pl.BlockSpec((4, BN), lambda i: (0, i)),
        ],
        out_shape=[
            jax.ShapeDtypeStruct((n_pad, D), jnp.float32),
            jax.ShapeDtypeStruct((n_pad, D), jnp.float32),
            jax.ShapeDtypeStruct((4, n_pad), jnp.float32),
        ],
    )


def _layer_body(relu_scale, acc_ref, norms_ref, w0, b0, w1, b1, *outs):
    norms = norms_ref[...]
    a0 = (acc_ref[0, 0] + acc_ref[0, 1]) * norms[1][:, None]
    a1 = (acc_ref[1, 0] + acc_ref[1, 1]) * norms[3][:, None]
    hh = (jnp.dot(a0, w0[...], preferred_element_type=jnp.float32) + b0[...]
          + jnp.dot(a1, w1[...], preferred_element_type=jnp.float32) + b1[...])
    if relu_scale:
        hh = jnp.maximum(hh, 0.0)
        outs[0][...] = hh * norms[0][:, None]
        outs[1][...] = hh * norms[2][:, None]
    else:
        outs[0][...] = hh


@functools.cache
def _make_layer(n_pad: int, relu_scale: bool):
    g = n_pad // BN
    n_out = 2 if relu_scale else 1
    return pl.pallas_call(
        functools.partial(_layer_body, relu_scale),
        grid=(g,),
        in_specs=[
            pl.BlockSpec((2, NC, BN, D), lambda i: (0, 0, i, 0)),
            pl.BlockSpec((4, BN), lambda i: (0, i)),
            pl.BlockSpec((D, D), lambda i: (0, 0)),
            pl.BlockSpec((1, D), lambda i: (0, 0)),
            pl.BlockSpec((D, D), lambda i: (0, 0)),
            pl.BlockSpec((1, D), lambda i: (0, 0)),
        ],
        out_specs=[pl.BlockSpec((BN, D), lambda i: (i, 0))] * n_out,
        out_shape=[jax.ShapeDtypeStruct((n_pad, D), jnp.float32)] * n_out,
    )


def kernel(x, edge_index_rel0, edge_index_rel1, neg_edge_index,
           W1_rel0, b1_rel0, W1_rel1, b1_rel1,
           W2_rel0, b2_rel0, W2_rel1, b2_rel1):
    n = x.shape[0]
    e = edge_index_rel0.shape[1]
    grain = NS * K * 8
    e_pad = -(-e // grain) * grain
    n_pad = -(-(n + 1) // (NS * 8 * 16)) * (NS * 8 * 16)

    def pad_edges(ei):
        fill = jnp.full((2, e_pad - e), n, jnp.int32)
        ei = jnp.concatenate([ei.astype(jnp.int32), fill], axis=1)
        return ei[0], ei[1]

    s0k, d0k = pad_edges(edge_index_rel0)
    s1k, d1k = pad_edges(edge_index_rel1)
    qsk, qdk = pad_edges(neg_edge_index)
    x_pad = jnp.pad(x, ((0, n_pad - n), (0, 0)))
    zrows = jnp.zeros((n_pad // NS, D), jnp.float32)
    zvec = jnp.zeros((n_pad,), jnp.float32)
    b10 = b1_rel0.reshape(1, D)
    b11 = b1_rel1.reshape(1, D)
    b20 = b2_rel0.reshape(1, D)
    b21 = b2_rel1.reshape(1, D)

    degp = _make_deg(n_pad, e_pad)(s0k, d0k, s1k, d1k, zvec)
    xs0, xs1, norms = _make_prep(n_pad)(x_pad, degp)
    conv2 = _make_conv2(n_pad, e_pad)
    acc_l1 = conv2(xs0, xs1, s0k, d0k, s1k, d1k, zrows)
    h1s0, h1s1 = _make_layer(n_pad, True)(
        acc_l1, norms, W1_rel0, b10, W1_rel1, b11)
    acc_l2 = conv2(h1s0, h1s1, s0k, d0k, s1k, d1k, zrows)
    (h,) = _make_layer(n_pad, False)(
        acc_l2, norms, W2_rel0, b20, W2_rel1, b21)
    scores = _make_score(n_pad, e_pad)(h, s0k, d0k, qsk, qdk)
    scores = scores.reshape(2, e_pad)
    return scores[0, :e], scores[1, :e]


# speedup vs baseline: 1.1765x; 1.1765x over previous
import functools

import jax
import jax.numpy as jnp
from jax import lax
from jax.experimental import pallas as pl
from jax.experimental.pallas import tpu as pltpu
from jax.experimental.pallas import tpu_sc as plsc

NC = 2
NS = 16
NW = NC * NS
L = 16
K = 128
D = 128
BN = 1024


def _mesh():
    return plsc.VectorSubcoreMesh(
        core_axis_name="c", subcore_axis_name="s", num_cores=NC, num_subcores=NS
    )


def _sc_params():
    return pltpu.CompilerParams(needs_layout_passes=False)


@functools.cache
def _make_deg(n_pad: int, e_pad: int):
    ep = e_pad // NW

    def body(s0, d0, s1, d1, zvec, out, idx, h0, h1, h2, h3):
        c = lax.axis_index("c")
        s = lax.axis_index("s")
        w = s * NC + c
        for h in (h0, h1, h2, h3):
            pltpu.sync_copy(zvec, h)
        ones = jnp.ones((L,), jnp.float32)
        for j, (arr, h) in enumerate(((s0, h0), (d0, h1), (s1, h2), (d1, h3))):
            def chunk(i, carry, arr=arr, h=h):
                pltpu.sync_copy(arr.at[pl.ds(w * ep + i * K, K)], idx)
                for g in range(K // L):
                    iv = idx[pl.ds(g * L, L)]
                    plsc.addupdate_scatter(h, [iv], ones)
                return carry
            lax.fori_loop(0, ep // K, chunk, 0)
            pltpu.sync_copy(h, out.at[j, w])

    return pl.kernel(
        body,
        out_type=jax.ShapeDtypeStruct((4, NW, n_pad), jnp.float32),
        mesh=_mesh(),
        compiler_params=_sc_params(),
        scratch_types=[
            pltpu.VMEM((K,), jnp.int32),
            pltpu.VMEM((n_pad,), jnp.float32),
            pltpu.VMEM((n_pad,), jnp.float32),
            pltpu.VMEM((n_pad,), jnp.float32),
            pltpu.VMEM((n_pad,), jnp.float32),
        ],
    )


def _split_chunks(total_chunks_per_pair: int, frac: float):
    n0 = int(round(total_chunks_per_pair * frac / 2)) * 2
    n0 = max(2, min(total_chunks_per_pair - 2, n0))
    return n0, total_chunks_per_pair - n0


@functools.cache
def _make_conv2(n_pad: int, e_pad: int):
    tch = e_pad // (NS * K)
    n0, n1 = _split_chunks(tch, 0.8625)
    rpt = n_pad // NS

    def body(xs0, xs1, src0, dst0, src1, dst1, zrows, out,
             s_a, d_a, s_b, d_b, rows_a, rows_b, acc,
             isem_a, isem_b, gsem_a, gsem_b):
        c = lax.axis_index("c")
        s = lax.axis_index("s")

        def wait_idx(sb, db, sem, src):
            pltpu.make_async_copy(src.at[pl.ds(0, K)], sb, sem).wait()
            pltpu.make_async_copy(src.at[pl.ds(0, K)], db, sem).wait()

        def pipeline(base, np_, xs, src, dst):
            pltpu.sync_copy(src.at[pl.ds(base, K)], s_a)
            pltpu.sync_copy(dst.at[pl.ds(base, K)], d_a)
            pltpu.async_copy(xs.at[s_a], rows_a, gsem_a)
            pltpu.async_copy(src.at[pl.ds(base + K, K)], s_b, isem_b)
            pltpu.async_copy(dst.at[pl.ds(base + K, K)], d_b, isem_b)

            def pair(p, carry):
                off2 = base + 2 * p * K
                wait_idx(s_b, d_b, isem_b, src)
                pltpu.async_copy(xs.at[s_b], rows_b, gsem_b)
                pltpu.make_async_copy(
                    xs.at[pl.ds(0, K)], rows_a, gsem_a).wait()
                pltpu.sync_copy(rows_a, acc.at[d_a], add=True)

                @pl.when(p < np_ - 1)
                def _():
                    pltpu.async_copy(src.at[pl.ds(off2 + 2 * K, K)],
                                     s_a, isem_a)
                    pltpu.async_copy(dst.at[pl.ds(off2 + 2 * K, K)],
                                     d_a, isem_a)

                pltpu.make_async_copy(
                    xs.at[pl.ds(0, K)], rows_b, gsem_b).wait()

                @pl.when(p < np_ - 1)
                def _():
                    wait_idx(s_a, d_a, isem_a, src)
                    pltpu.async_copy(xs.at[s_a], rows_a, gsem_a)

                pltpu.sync_copy(rows_b, acc.at[d_b], add=True)

                @pl.when(p < np_ - 1)
                def _():
                    pltpu.async_copy(src.at[pl.ds(off2 + 3 * K, K)],
                                     s_b, isem_b)
                    pltpu.async_copy(dst.at[pl.ds(off2 + 3 * K, K)],
                                     d_b, isem_b)

                return carry

            lax.fori_loop(0, np_, pair, 0)

        def phase(r, xs, src, dst):
            plsc.subcore_barrier()

            @pl.when(c == 0)
            def _():
                pipeline(s * (n0 * K), n0 // 2, xs, src, dst)

            @pl.when(c == 1)
            def _():
                pipeline(NS * n0 * K + s * (n1 * K), n1 // 2, xs, src, dst)

            plsc.subcore_barrier()
            pltpu.sync_copy(acc.at[pl.ds(s * rpt, rpt)],
                            out.at[r, c, pl.ds(s * rpt, rpt)])
            if r == 0:
                pltpu.sync_copy(zrows, acc.at[pl.ds(s * rpt, rpt)])

        pltpu.sync_copy(zrows, acc.at[pl.ds(s * rpt, rpt)])
        phase(0, xs0, src0, dst0)
        phase(1, xs1, src1, dst1)

    return pl.kernel(
        body,
        out_type=jax.ShapeDtypeStruct((2, NC, n_pad, D), jnp.float32),
        mesh=_mesh(),
        compiler_params=_sc_params(),
        scratch_types=[
            pltpu.VMEM((K,), jnp.int32),
            pltpu.VMEM((K,), jnp.int32),
            pltpu.VMEM((K,), jnp.int32),
            pltpu.VMEM((K,), jnp.int32),
            pltpu.VMEM((K, D), jnp.float32),
            pltpu.VMEM((K, D), jnp.float32),
            pltpu.VMEM_SHARED((n_pad, D), jnp.float32),
            pltpu.SemaphoreType.DMA,
            pltpu.SemaphoreType.DMA,
            pltpu.SemaphoreType.DMA,
            pltpu.SemaphoreType.DMA,
        ],
    )


@functools.cache
def _make_score(n_pad: int, e_pad: int):
    tch = e_pad // (NS * K)
    m0, m1 = _split_chunks(tch, 0.9)
    if tch % 8 == 0:
        m0 = max(8, (m0 // 8) * 8)
        m1 = tch - m0

    def body(h, ps, pd, qs, qd, out, s_a, d_a, s_b, d_b,
             srows_a, drows_a, srows_b, drows_b, p16, dots,
             isem_a, isem_b, gs_a, gd_a, gs_b, gd_b):
        c = lax.axis_index("c")
        s = lax.axis_index("s")
        lane = lax.iota(jnp.int32, L)

        def compute(ci, srows, drows):
            def row(r, carry2):
                acc = srows[r, pl.ds(0, L)] * drows[r, pl.ds(0, L)]
                for jj in range(1, D // L):
                    acc = acc + (srows[r, pl.ds(jj * L, L)]
                                 * drows[r, pl.ds(jj * L, L)])
                p16[r, :] = acc
                return carry2

            lax.fori_loop(0, K, row, 0)
            for g in range(K // L):
                rowi = lane + (g * L)
                dv = plsc.load_gather(
                    p16, [rowi, jnp.zeros((L,), jnp.int32)])
                for col in range(1, L):
                    dv = dv + plsc.load_gather(
                        p16, [rowi, jnp.full((L,), col, jnp.int32)])
                dots[ci, pl.ds(g * L, L)] = dv

        def wait_idx(sb, db, sem, sa):
            pltpu.make_async_copy(sa.at[pl.ds(0, K)], sb, sem).wait()
            pltpu.make_async_copy(sa.at[pl.ds(0, K)], db, sem).wait()

        def wait_rows(srows, drows, sem_s, sem_d):
            pltpu.make_async_copy(h.at[pl.ds(0, K)], srows, sem_s).wait()
            pltpu.make_async_copy(h.at[pl.ds(0, K)], drows, sem_d).wait()

        def pipeline(base, np_, sa, da):
            pltpu.sync_copy(sa.at[pl.ds(base, K)], s_a)
            pltpu.sync_copy(da.at[pl.ds(base, K)], d_a)
            pltpu.async_copy(h.at[s_a], srows_a, gs_a)
            pltpu.async_copy(h.at[d_a], drows_a, gd_a)
            pltpu.async_copy(sa.at[pl.ds(base + K, K)], s_b, isem_b)
            pltpu.async_copy(da.at[pl.ds(base + K, K)], d_b, isem_b)

            def pair(p, carry):
                c0 = 2 * p
                off2 = base + 2 * p * K
                wait_idx(s_b, d_b, isem_b, sa)
                pltpu.async_copy(h.at[s_b], srows_b, gs_b)
                pltpu.async_copy(h.at[d_b], drows_b, gd_b)
                wait_rows(srows_a, drows_a, gs_a, gd_a)
                compute(c0, srows_a, drows_a)

                @pl.when(p < np_ - 1)
                def _():
                    pltpu.async_copy(sa.at[pl.ds(off2 + 2 * K, K)],
                                     s_a, isem_a)
                    pltpu.async_copy(da.at[pl.ds(off2 + 2 * K, K)],
                                     d_a, isem_a)

                wait_rows(srows_b, drows_b, gs_b, gd_b)

                @pl.when(p < np_ - 1)
                def _():
                    wait_idx(s_a, d_a, isem_a, sa)
                    pltpu.async_copy(h.at[s_a], srows_a, gs_a)
                    pltpu.async_copy(h.at[d_a], drows_a, gd_a)

                compute(c0 + 1, srows_b, drows_b)

                @pl.when(p < np_ - 1)
                def _():
                    pltpu.async_copy(sa.at[pl.ds(off2 + 3 * K, K)],
                                     s_b, isem_b)
                    pltpu.async_copy(da.at[pl.ds(off2 + 3 * K, K)],
                                     d_b, isem_b)

                return carry

            lax.fori_loop(0, np_, pair, 0)

        for which, (sa, da) in enumerate(((ps, pd), (qs, qd))):

            @pl.when(c == 0)
            def _(sa=sa, da=da, which=which):
                pipeline(s * (m0 * K), m0 // 2, sa, da)
                cb = pl.multiple_of(s * m0, 8)
                pltpu.sync_copy(dots.at[pl.ds(0, m0)],
                                out.at[which, pl.ds(cb, m0)])

            @pl.when(c == 1)
            def _(sa=sa, da=da, which=which):
                pipeline(NS * m0 * K + s * (m1 * K), m1 // 2, sa, da)
                cb = pl.multiple_of(NS * m0 + s * m1, 8)
                pltpu.sync_copy(dots.at[pl.ds(0, m1)],
                                out.at[which, pl.ds(cb, m1)])

    return pl.kernel(
        body,
        out_type=jax.ShapeDtypeStruct((2, e_pad // K, K), jnp.float32),
        mesh=_mesh(),
        compiler_params=_sc_params(),
        scratch_types=[
            pltpu.VMEM((K,), jnp.int32),
            pltpu.VMEM((K,), jnp.int32),
            pltpu.VMEM((K,), jnp.int32),
            pltpu.VMEM((K,), jnp.int32),
            pltpu.VMEM((K, D), jnp.float32),
            pltpu.VMEM((K, D), jnp.float32),
            pltpu.VMEM((K, D), jnp.float32),
            pltpu.VMEM((K, D), jnp.float32),
            pltpu.VMEM((K, L), jnp.float32),
            pltpu.VMEM((m0, K), jnp.float32),
            pltpu.SemaphoreType.DMA,
            pltpu.SemaphoreType.DMA,
            pltpu.SemaphoreType.DMA,
            pltpu.SemaphoreType.DMA,
            pltpu.SemaphoreType.DMA,
            pltpu.SemaphoreType.DMA,
        ],
    )


def _prep_body(x_ref, degp_ref, xs0_ref, xs1_ref, norms_ref):
    deg = jnp.sum(degp_ref[...], axis=1)
    norms = jnp.where(deg > 0, lax.rsqrt(jnp.maximum(deg, 1.0)), 0.0)
    norms_ref[...] = norms
    x = x_ref[...]
    xs0_ref[...] = x * norms[0][:, None]
    xs1_ref[...] = x * norms[2][:, None]


@functools.cache
def _make_prep(n_pad: int):
    g = n_pad // BN
    return pl.pallas_call(
        _prep_body,
        grid=(g,),
        in_specs=[
            pl.BlockSpec((BN, D), lambda i: (i, 0)),
            pl.BlockSpec((4, NW, BN), lambda i: (0, 0, i)),
        ],
        out_specs=[
            pl.BlockSpec((BN, D), lambda i: (i, 0)),
            pl.BlockSpec((BN, D), lambda i: (i, 0)),
            pl.BlockSpec((4, BN), lambda i: (0, i)),
        ],
        out_shape=[
            jax.ShapeDtypeStruct((n_pad, D), jnp.float32),
            jax.ShapeDtypeStruct((n_pad, D), jnp.float32),
            jax.ShapeDtypeStruct((4, n_pad), jnp.float32),
        ],
    )


def _layer_body(relu_scale, acc_ref, norms_ref, w0, b0, w1, b1, *outs):
    norms = norms_ref[...]
    a0 = (acc_ref[0, 0] + acc_ref[0, 1]) * norms[1][:, None]
    a1 = (acc_ref[1, 0] + acc_ref[1, 1]) * norms[3][:, None]
    hh = (jnp.dot(a0, w0[...], preferred_element_type=jnp.float32) + b0[...]
          + jnp.dot(a1, w1[...], preferred_element_type=jnp.float32) + b1[...])
    if relu_scale:
        hh = jnp.maximum(hh, 0.0)
        outs[0][...] = hh * norms[0][:, None]
        outs[1][...] = hh * norms[2][:, None]
    else:
        outs[0][...] = hh


@functools.cache
def _make_layer(n_pad: int, relu_scale: bool):
    g = n_pad // BN
    n_out = 2 if relu_scale else 1
    return pl.pallas_call(
        functools.partial(_layer_body, relu_scale),
        grid=(g,),
        in_specs=[
            pl.BlockSpec((2, NC, BN, D), lambda i: (0, 0, i, 0)),
            pl.BlockSpec((4, BN), lambda i: (0, i)),
            pl.BlockSpec((D, D), lambda i: (0, 0)),
            pl.BlockSpec((1, D), lambda i: (0, 0)),
            pl.BlockSpec((D, D), lambda i: (0, 0)),
            pl.BlockSpec((1, D), lambda i: (0, 0)),
        ],
        out_specs=[pl.BlockSpec((BN, D), lambda i: (i, 0))] * n_out,
        out_shape=[jax.ShapeDtypeStruct((n_pad, D), jnp.float32)] * n_out,
    )


def kernel(x, edge_index_rel0, edge_index_rel1, neg_edge_index,
           W1_rel0, b1_rel0, W1_rel1, b1_rel1,
           W2_rel0, b2_rel0, W2_rel1, b2_rel1):
    n = x.shape[0]
    e = edge_index_rel0.shape[1]
    grain = NS * K * 8
    e_pad = -(-e // grain) * grain
    n_pad = -(-(n + 1) // (NS * 8 * 16)) * (NS * 8 * 16)

    def pad_edges(ei):
        fill = jnp.full((2, e_pad - e), n, jnp.int32)
        ei = jnp.concatenate([ei.astype(jnp.int32), fill], axis=1)
        return ei[0], ei[1]

    s0k, d0k = pad_edges(edge_index_rel0)
    s1k, d1k = pad_edges(edge_index_rel1)
    qsk, qdk = pad_edges(neg_edge_index)
    x_pad = jnp.pad(x, ((0, n_pad - n), (0, 0)))
    zrows = jnp.zeros((n_pad // NS, D), jnp.float32)
    zvec = jnp.zeros((n_pad,), jnp.float32)
    b10 = b1_rel0.reshape(1, D)
    b11 = b1_rel1.reshape(1, D)
    b20 = b2_rel0.reshape(1, D)
    b21 = b2_rel1.reshape(1, D)

    degp = _make_deg(n_pad, e_pad)(s0k, d0k, s1k, d1k, zvec)
    xs0, xs1, norms = _make_prep(n_pad)(x_pad, degp)
    conv2 = _make_conv2(n_pad, e_pad)
    acc_l1 = conv2(xs0, xs1, s0k, d0k, s1k, d1k, zrows)
    h1s0, h1s1 = _make_layer(n_pad, True)(
        acc_l1, norms, W1_rel0, b10, W1_rel1, b11)
    acc_l2 = conv2(h1s0, h1s1, s0k, d0k, s1k, d1k, zrows)
    (h,) = _make_layer(n_pad, False)(
        acc_l2, norms, W2_rel0, b20, W2_rel1, b21)
    scores = _make_score(n_pad, e_pad)(h, s0k, d0k, qsk, qdk)
    scores = scores.reshape(2, e_pad)
    return scores[0, :e], scores[1, :e]
